# baseline (speedup 1.0000x reference)
import functools

import jax
import jax.numpy as jnp
from jax import lax
from jax.experimental import pallas as pl
from jax.experimental.pallas import tpu as pltpu
from jax.experimental.pallas import tpu_sc as plsc

NC = 2
NS = 16
NW = NC * NS
CH_E = 104
CH_O = 96
NBUF = 4
MAX_NORM = 1.0
EPS = 1e-7


def _renorm_chunk(buf, n_rows, d, grp):
    n_slices = d // 16
    lanes = lax.iota(jnp.int32, 16)

    def group_body(gi, _):
        r0 = gi * grp
        parts = []
        accs = []
        for j in range(grp):
            p = [buf[r0 + j, pl.ds(16 * c, 16)] for c in range(n_slices)]
            parts.append(p)
            sv = p[0] * p[0]
            for c in range(1, n_slices):
                sv = sv + p[c] * p[c]
            for sh in (8, 4, 2, 1):
                sv = sv + sv.at[lanes ^ sh].get(mode="promise_in_bounds")
            accs.append(jnp.where(lanes == j, sv, 0.0))
        while len(accs) > 1:
            nxt = [accs[k] + accs[k + 1] for k in range(0, len(accs) - 1, 2)]
            if len(accs) % 2:
                nxt[-1] = nxt[-1] + accs[-1]
            accs = nxt
        acc = accs[0]
        iv = lax.bitcast_convert_type(acc, jnp.int32)
        iv = jnp.int32(0x5F3759DF) - (iv >> 1)
        y = lax.bitcast_convert_type(iv, jnp.float32)
        half = 0.5 * acc
        for _ in range(3):
            y = y * (1.5 - half * y * y)
        scale = jnp.where(acc > MAX_NORM * MAX_NORM, y * (1.0 - EPS * y), 1.0)
        for j in range(grp):
            sj = scale.at[jnp.full((16,), j, jnp.int32)].get(
                mode="promise_in_bounds")
            for c in range(n_slices):
                buf[r0 + j, pl.ds(16 * c, 16)] = parts[j][c] * sj
        return 0

    lax.fori_loop(0, n_rows // grp, group_body, 0)


def kernel(input, speakers, lut_p, lut_s):
    B, L = input.shape
    V, D = lut_p.shape
    rows_per_w = B // NW
    n_chunks = rows_per_w * 2
    spk_per_w = B // NW
    assert rows_per_w * NW == B and CH_E + CH_O == L
    assert spk_per_w <= 128 and n_chunks % NBUF == 0 and n_chunks >= 3 * NBUF

    idx = input.astype(jnp.int32)
    spk = speakers.astype(jnp.int32)

    mesh = plsc.VectorSubcoreMesh(core_axis_name="c", subcore_axis_name="s")

    @functools.partial(
        pl.kernel,
        mesh=mesh,
        compiler_params=pltpu.CompilerParams(use_tc_tiling_on_sc=False),
        out_type=[
            jax.ShapeDtypeStruct((B, L, D), jnp.float32),
            jax.ShapeDtypeStruct((B, D), jnp.float32),
        ],
        scratch_types=[
            pltpu.VMEM((rows_per_w, L), jnp.int32),
            pltpu.VMEM((spk_per_w,), jnp.int32),
            pltpu.VMEM((NBUF, CH_E, D), jnp.float32),
            pltpu.VMEM((spk_per_w, D), jnp.float32),
            pltpu.SemaphoreType.DMA((NBUF,)),
            pltpu.SemaphoreType.DMA((NBUF,)),
            pltpu.SemaphoreType.DMA,
        ],
    )
    def run(idx_hbm, spk_hbm, lut_p_hbm, lut_s_hbm, out_hbm, ident_hbm,
            idx_v, spk_v, rows_v, srows_v, gsem, osem, ssem):
        cid = lax.axis_index("c")
        sid = lax.axis_index("s")
        wid = sid * NC + cid
        row0 = wid * rows_per_w

        pltpu.sync_copy(idx_hbm.at[pl.ds(row0, rows_per_w)], idx_v)
        pltpu.sync_copy(spk_hbm.at[pl.ds(wid * spk_per_w, spk_per_w)], spk_v)

        pltpu.async_copy(lut_s_hbm.at[spk_v], srows_v, ssem).wait()
        _renorm_chunk(srows_v, spk_per_w, D, 8)
        pltpu.sync_copy(
            srows_v, ident_hbm.at[pl.ds(wid * spk_per_w, spk_per_w)])

        def chunk_dims(k_static_parity):
            if k_static_parity == 0:
                return CH_E, 0
            return CH_O, CH_E

        def idx_slice(k, par):
            sz, off = chunk_dims(par)
            return idx_v.at[k >> 1, pl.ds(off, sz)]

        def out_slice(k, par):
            sz, off = chunk_dims(par)
            return out_hbm.at[row0 + (k >> 1), pl.ds(off, sz)]

        def buf_slice(p, par):
            sz, _ = chunk_dims(par)
            return rows_v.at[p, pl.ds(0, sz)]

        def start_gather(k, p):
            par = p % 2
            pltpu.async_copy(lut_p_hbm.at[idx_slice(k, par)],
                             buf_slice(p, par), gsem.at[p])

        def wait_gather(k, p):
            par = p % 2
            pltpu.make_async_copy(lut_p_hbm.at[idx_slice(k, par)],
                                  buf_slice(p, par), gsem.at[p]).wait()

        def start_write(k, p):
            par = p % 2
            pltpu.async_copy(buf_slice(p, par), out_slice(k, par),
                             osem.at[p])

        def wait_write(k, p):
            par = p % 2
            pltpu.make_async_copy(buf_slice(p, par), out_slice(k, par),
                                  osem.at[p]).wait()

        def body(k, p, first, last):
            wait_gather(k, p)
            _renorm_chunk(rows_v.at[p], chunk_dims(p % 2)[0], D, 8)
            start_write(k, p)
            h = k + 2
            q = (p + 2) % NBUF
            if not last:
                if not first:
                    wait_write(h - NBUF, q)
                start_gather(h, q)

        start_gather(jnp.int32(0), 0)
        start_gather(jnp.int32(1), 1)

        for p in range(NBUF):
            body(jnp.int32(p), p, first=(p < 2), last=False)

        def round_body(i, _):
            k0 = i * NBUF
            for p in range(NBUF):
                body(k0 + p, p, first=False, last=False)
            return 0

        lax.fori_loop(1, n_chunks // NBUF - 1, round_body, 0)

        k0 = n_chunks - NBUF
        for p in range(NBUF):
            body(jnp.int32(k0 + p), p, first=False, last=(p >= 2))

        for p in range(NBUF):
            wait_write(jnp.int32(k0 + p), p)

    out, ident = run(idx, spk, lut_p, lut_s)
    return out, ident
